# baseline (speedup 1.0000x reference)
import jax
import jax.numpy as jnp
from jax.experimental import pallas as pl
from jax.experimental.pallas import tpu as pltpu
from jax.experimental.pallas import tpu_sc as plsc

BATCH = 16384
NUM_CLASSES = 1000
NUM_WORKERS = 32
ROWS_PER_WORKER = BATCH // NUM_WORKERS
CHUNK_ROWS = 16
NUM_BUFS = 4
CHUNKS_PER_WORKER = ROWS_PER_WORKER // CHUNK_ROWS

_mesh = plsc.VectorSubcoreMesh(
    core_axis_name="c", subcore_axis_name="s", num_cores=2
)


@pl.kernel(
    mesh=_mesh,
    out_type=jax.ShapeDtypeStruct((BATCH, NUM_CLASSES), jnp.float32),
    scratch_types=[
        pltpu.VMEM((ROWS_PER_WORKER,), jnp.int32),
        [pltpu.VMEM((CHUNK_ROWS, NUM_CLASSES), jnp.float32)] * NUM_BUFS,
        [pltpu.SemaphoreType.DMA] * NUM_BUFS,
    ],
    compiler_params=pltpu.CompilerParams(needs_layout_passes=False),
)
def _onehot_sc(x_hbm, zblk_hbm, out_hbm, xs, bufs, sems):
    wid = jax.lax.axis_index("s") * 2 + jax.lax.axis_index("c")
    base = wid * ROWS_PER_WORKER
    pltpu.sync_copy(x_hbm.at[pl.ds(base, ROWS_PER_WORKER)], xs)
    for b in range(NUM_BUFS):
        pltpu.sync_copy(zblk_hbm, bufs[b])

    lane = jax.lax.iota(jnp.int32, 16)
    ones = jnp.full((16,), 1.0, jnp.float32)
    zeros = jnp.full((16,), 0.0, jnp.float32)

    def _scatter(buf, ci, vals):
        cols = xs[pl.ds(ci * CHUNK_ROWS, CHUNK_ROWS)]
        plsc.store_scatter(buf, [lane, cols], vals)

    def _step(buf, sem, ci, k):
        @pl.when(k >= 1)
        def _drain():
            pltpu.make_async_copy(
                buf, out_hbm.at[pl.ds(0, CHUNK_ROWS)], sem
            ).wait()
            _scatter(buf, ci - NUM_BUFS, zeros)

        _scatter(buf, ci, ones)
        pltpu.make_async_copy(
            buf, out_hbm.at[pl.ds(base + ci * CHUNK_ROWS, CHUNK_ROWS)], sem
        ).start()

    def _body(k, carry):
        for b in range(NUM_BUFS):
            _step(bufs[b], sems[b], NUM_BUFS * k + b, k)
        return carry

    jax.lax.fori_loop(0, CHUNKS_PER_WORKER // NUM_BUFS, _body, 0)
    for b in range(NUM_BUFS):
        pltpu.make_async_copy(
            bufs[b], out_hbm.at[pl.ds(0, CHUNK_ROWS)], sems[b]
        ).wait()


def kernel(x):
    xf = x.astype(jnp.int32).reshape(BATCH)
    zblk = jnp.zeros((CHUNK_ROWS, NUM_CLASSES), jnp.float32)
    return _onehot_sc(xf, zblk)


# speedup vs baseline: 1.1045x; 1.1045x over previous
import jax
import jax.numpy as jnp
from jax.experimental import pallas as pl

BATCH = 16384
NUM_CLASSES = 1000
ROW_BLOCK = 2048


def _onehot_block(x_ref, o_ref):
    idx = x_ref[...]
    cols = jax.lax.broadcasted_iota(jnp.int32, o_ref.shape, 1)
    o_ref[...] = (cols == idx).astype(jnp.int8)


def kernel(x):
    x = x.astype(jnp.int32)
    oh8 = pl.pallas_call(
        _onehot_block,
        grid=(BATCH // ROW_BLOCK,),
        in_specs=[pl.BlockSpec((ROW_BLOCK, 1), lambda i: (i, 0))],
        out_specs=pl.BlockSpec((ROW_BLOCK, NUM_CLASSES), lambda i: (i, 0)),
        out_shape=jax.ShapeDtypeStruct((BATCH, NUM_CLASSES), jnp.int8),
    )(x)
    return oh8.astype(jnp.float32)
